# baseline (speedup 1.0000x reference)
import jax
import jax.numpy as jnp
from jax.experimental import pallas as pl
from jax.experimental.pallas import tpu as pltpu

B, L, N, F, H, K = 2, 8, 1024, 128, 128, 5
BL = B * L


def _dot(a, b):
    return jnp.dot(a, b, preferred_element_type=jnp.float32)


def _kth_largest_exact(sim):
    neg = jnp.float32(-jnp.inf)
    cur = jnp.full((N, 1), jnp.inf, dtype=jnp.float32)
    remaining = jnp.full((N, 1), K, dtype=jnp.int32)
    thresh = jnp.full((N, 1), neg, dtype=jnp.float32)
    done = jnp.zeros((N, 1), dtype=jnp.bool_)
    for _ in range(K):
        masked = jnp.where(sim < cur, sim, neg)
        m = jnp.max(masked, axis=1, keepdims=True)
        c = jnp.sum((sim == m).astype(jnp.int32), axis=1, keepdims=True)
        newly = jnp.logical_and(jnp.logical_not(done), c >= remaining)
        thresh = jnp.where(newly, m, thresh)
        remaining = remaining - jnp.where(done, 0, c)
        done = jnp.logical_or(done, newly)
        cur = m
    return thresh


def _local_thresh_and_absdeg(sim):
    neg = jnp.float32(-jnp.inf)
    ds = [jnp.max(sim, axis=1, keepdims=True)]
    for _ in range(K - 1):
        ds.append(jnp.max(jnp.where(sim < ds[-1], sim, neg), axis=1,
                          keepdims=True))
    d = ds[-1]
    cnt = jnp.sum(jnp.where(sim >= d, 1.0, 0.0), axis=1, keepdims=True)
    has_ties = jnp.any(cnt != jnp.float32(K))

    def _exact():
        t = _kth_largest_exact(sim)
        s = jnp.sum(jnp.where(sim >= t, jnp.abs(sim), 0.0), axis=1,
                    keepdims=True)
        return t, s

    def _fast():
        return d, sum(jnp.abs(v) for v in ds)

    return jax.lax.cond(has_ties, _exact, _fast)


def _snapshot_kernel(x_ref, w0_ref, b0_ref, w1_ref, b1_ref, wg0_ref, bg0_ref,
                     wg1_ref, bg1_ref, wn0_ref, bn0_ref, wn1_ref, bn1_ref,
                     out_ref):
    for s in range(2):
        _one_snapshot(x_ref, w0_ref, b0_ref, w1_ref, b1_ref, wg0_ref, bg0_ref,
                      wg1_ref, bg1_ref, wn0_ref, bn0_ref, wn1_ref, bn1_ref,
                      out_ref, s)


def _one_snapshot(x_ref, w0_ref, b0_ref, w1_ref, b1_ref, wg0_ref, bg0_ref,
                  wg1_ref, bg1_ref, wn0_ref, bn0_ref, wn1_ref, bn1_ref,
                  out_ref, s):
    x = x_ref[s]
    sim = jax.lax.dot_general(x, x, (((1,), (1,)), ((), ())),
                              preferred_element_type=jnp.float32)

    dis_g = jax.lax.rsqrt(
        jnp.sum(jnp.abs(sim), axis=1, keepdims=True) + (1.0 + 1e-6))

    thresh, absdeg = _local_thresh_and_absdeg(sim)
    local = jnp.where(sim >= thresh, sim, 0.0)
    dis_l = jax.lax.rsqrt(absdeg + (1.0 + 1e-6))

    w1 = w1_ref[...]
    b0 = b0_ref[...]
    b1 = b1_ref[...]
    z0 = _dot(x, w0_ref[...])

    def gcn(a, dis):
        u = z0 * dis
        h1 = jax.nn.relu((_dot(a, u) + u) * dis + b0) + x
        u2 = _dot(h1, w1) * dis
        h2 = jax.nn.relu((_dot(a, u2) + u2) * dis + b1) + h1
        return h2

    hg = gcn(sim, dis_g)
    hl = gcn(local, dis_l)

    wg0 = wg0_ref[...]
    u = jax.nn.relu(_dot(hg, wg0[:H]) + _dot(hl, wg0[H:]) + bg0_ref[...])
    cw = _dot(u, wg1_ref[...]) + bg1_ref[...]

    t = jax.lax.dot_general(cw, wn0_ref[...], (((0,), (0,)), ((), ())),
                            preferred_element_type=jnp.float32)
    z3 = jax.nn.relu(t + bn0_ref[...])
    out = jnp.sum(z3 * wn1_ref[...], axis=1) + bn1_ref[0, 0]
    out_ref[s, 0, :] = out


def kernel(node_features, W0, b0, W1, b1, Wg0, bg0, Wg1, bg1, Wn0, bn0, Wn1,
           bn1):
    x = node_features.reshape(BL, N, F)
    row = lambda v: v.reshape(1, -1)
    full = lambda s: pl.BlockSpec(s, lambda i: (0,) * len(s))
    out = pl.pallas_call(
        _snapshot_kernel,
        grid=(BL // 2,),
        in_specs=[
            pl.BlockSpec((2, N, F), lambda i: (i, 0, 0)),
            full((F, H)), full((1, H)),
            full((H, H)), full((1, H)),
            full((2 * H, H)), full((1, H)),
            full((H, H)), full((1, H)),
            full((N, H)), full((1, H)),
            full((1, H)), full((1, 1)),
        ],
        out_specs=pl.BlockSpec((2, 1, H), lambda i: (i, 0, 0)),
        out_shape=jax.ShapeDtypeStruct((BL, 1, H), jnp.float32),
        compiler_params=pltpu.CompilerParams(
            dimension_semantics=("parallel",)),
    )(x, W0, row(b0), W1, row(b1), Wg0, row(bg0), Wg1, row(bg1), Wn0,
      row(bn0), Wn1.reshape(1, H), bn1.reshape(1, 1))
    return out.reshape(B, L, H)


# speedup vs baseline: 1.0609x; 1.0609x over previous
import jax
import jax.numpy as jnp
from jax.experimental import pallas as pl
from jax.experimental.pallas import tpu as pltpu

B, L, N, F, H, K = 2, 8, 1024, 128, 128, 5
BL = B * L


def _dot(a, b):
    return jnp.dot(a, b, preferred_element_type=jnp.float32)


def _kth_largest_exact(sim):
    neg = jnp.float32(-jnp.inf)
    cur = jnp.full((N, 1), jnp.inf, dtype=jnp.float32)
    remaining = jnp.full((N, 1), K, dtype=jnp.int32)
    thresh = jnp.full((N, 1), neg, dtype=jnp.float32)
    done = jnp.zeros((N, 1), dtype=jnp.bool_)
    for _ in range(K):
        masked = jnp.where(sim < cur, sim, neg)
        m = jnp.max(masked, axis=1, keepdims=True)
        c = jnp.sum((sim == m).astype(jnp.int32), axis=1, keepdims=True)
        newly = jnp.logical_and(jnp.logical_not(done), c >= remaining)
        thresh = jnp.where(newly, m, thresh)
        remaining = remaining - jnp.where(done, 0, c)
        done = jnp.logical_or(done, newly)
        cur = m
    return thresh


def _kth_largest_per_row(sim):
    neg = jnp.float32(-jnp.inf)
    d = jnp.max(sim, axis=1, keepdims=True)
    for _ in range(K - 1):
        d = jnp.max(jnp.where(sim < d, sim, neg), axis=1, keepdims=True)
    cnt = jnp.sum(jnp.where(sim >= d, 1.0, 0.0), axis=1, keepdims=True)
    has_ties = jnp.any(cnt != jnp.float32(K))
    return jax.lax.cond(has_ties, lambda: _kth_largest_exact(sim), lambda: d)


def _snapshot_kernel(x_ref, w0_ref, b0_ref, w1_ref, b1_ref, wg0_ref, bg0_ref,
                     wg1_ref, bg1_ref, wn0_ref, bn0_ref, wn1_ref, bn1_ref,
                     out_ref):
    for s in range(2):
        _one_snapshot(x_ref, w0_ref, b0_ref, w1_ref, b1_ref, wg0_ref, bg0_ref,
                      wg1_ref, bg1_ref, wn0_ref, bn0_ref, wn1_ref, bn1_ref,
                      out_ref, s)


def _one_snapshot(x_ref, w0_ref, b0_ref, w1_ref, b1_ref, wg0_ref, bg0_ref,
                  wg1_ref, bg1_ref, wn0_ref, bn0_ref, wn1_ref, bn1_ref,
                  out_ref, s):
    x = x_ref[s]
    sim = jax.lax.dot_general(x, x, (((1,), (1,)), ((), ())),
                              preferred_element_type=jnp.float32)

    dis_g = jax.lax.rsqrt(
        jnp.sum(jnp.abs(sim), axis=1, keepdims=True) + (1.0 + 1e-6))

    local = jnp.where(sim >= _kth_largest_per_row(sim), sim, 0.0)
    dis_l = jax.lax.rsqrt(
        jnp.sum(jnp.abs(local), axis=1, keepdims=True) + (1.0 + 1e-6))

    w1 = w1_ref[...]
    b0 = b0_ref[...]
    b1 = b1_ref[...]
    z0 = _dot(x, w0_ref[...])

    def gcn(a, dis):
        u = z0 * dis
        h1 = jax.nn.relu((_dot(a, u) + u) * dis + b0) + x
        u2 = _dot(h1, w1) * dis
        h2 = jax.nn.relu((_dot(a, u2) + u2) * dis + b1) + h1
        return h2

    hg = gcn(sim, dis_g)
    hl = gcn(local, dis_l)

    wg0 = wg0_ref[...]
    u = jax.nn.relu(_dot(hg, wg0[:H]) + _dot(hl, wg0[H:]) + bg0_ref[...])
    cw = _dot(u, wg1_ref[...]) + bg1_ref[...]

    t = jax.lax.dot_general(cw, wn0_ref[...], (((0,), (0,)), ((), ())),
                            preferred_element_type=jnp.float32)
    z3 = jax.nn.relu(t + bn0_ref[...])
    out = jnp.sum(z3 * wn1_ref[...], axis=1) + bn1_ref[0, 0]
    out_ref[s, 0, :] = out


def kernel(node_features, W0, b0, W1, b1, Wg0, bg0, Wg1, bg1, Wn0, bn0, Wn1,
           bn1):
    x = node_features.reshape(BL, N, F)
    row = lambda v: v.reshape(1, -1)
    full = lambda s: pl.BlockSpec(s, lambda i: (0,) * len(s))
    out = pl.pallas_call(
        _snapshot_kernel,
        grid=(BL // 2,),
        in_specs=[
            pl.BlockSpec((2, N, F), lambda i: (i, 0, 0)),
            full((F, H)), full((1, H)),
            full((H, H)), full((1, H)),
            full((2 * H, H)), full((1, H)),
            full((H, H)), full((1, H)),
            full((N, H)), full((1, H)),
            full((1, H)), full((1, 1)),
        ],
        out_specs=pl.BlockSpec((2, 1, H), lambda i: (i, 0, 0)),
        out_shape=jax.ShapeDtypeStruct((BL, 1, H), jnp.float32),
        compiler_params=pltpu.CompilerParams(
            dimension_semantics=("parallel",)),
    )(x, W0, row(b0), W1, row(b1), Wg0, row(bg0), Wg1, row(bg1), Wn0,
      row(bn0), Wn1.reshape(1, H), bn1.reshape(1, 1))
    return out.reshape(B, L, H)
